# baseline (speedup 1.0000x reference)
import functools

import jax
import jax.numpy as jnp
from jax import lax
from jax.experimental import pallas as pl
from jax.experimental.pallas import tpu as pltpu
from jax.experimental.pallas import tpu_sc as plsc

N = 10000
NP = 10240
E = 320000
HB = 65536
OBJ_D = 80
ATTR_DP = 64
HASH_D = 32
OUT_D = 128
HALF_D = 64

NC = 2
NS = 16
NW = NC * NS
NODES_W = NP // NW
EDGES_T = E // NS
CH = 80
ROWS_T = NP // NS
RING = 5
EGROUPS = EDGES_T // (CH * RING)
NSTAGE = 160

_F32 = jnp.float32


def _gelu(x):
    return x * 0.5 * (1.0 + lax.erf(x * 0.7071067811865476))


def _sc_body(obj_id_h, obj_hash_id_h, attr_idT_h, attr_hash_idT_h,
             dst_h, ehash2_h,
             obj_emb_h, attr_emb_h, obj_hash_emb_h, attr_hash_emb_h, gtab_h,
             zacc_h, zdeg_h,
             o_rows_h, av_rows_h, oh_rows_h, ah_rows_h, acc_out_h, deg_out_h,
             nidx, nhidx, aidx, ahidx,
             hidx0, hidx1, hidx2, hidx3, hidx4,
             dstb0, dstb1, dstb2, dstb3, dstb4,
             rows0, rows1, rows2, rows3, rows4,
             rA, rB, rC, ones_v, hblk0, hblk1, acc_sh, deg_sh,
             nsem, wsemA, wsemB, wsemC, hsem,
             gsem0, gsem1, gsem2, gsem3, gsem4,
             dsem0, dsem1, dsem2, dsem3, dsem4,
             ssem0, ssem1, ssem2, ssem3, ssem4):
    c = lax.axis_index("c")
    s = lax.axis_index("s")
    wid = c * NS + s
    hidx = (hidx0, hidx1, hidx2, hidx3, hidx4)
    dstb = (dstb0, dstb1, dstb2, dstb3, dstb4)
    rows = (rows0, rows1, rows2, rows3, rows4)
    gsem = (gsem0, gsem1, gsem2, gsem3, gsem4)
    dsem = (dsem0, dsem1, dsem2, dsem3, dsem4)
    ssem = (ssem0, ssem1, ssem2, ssem3, ssem4)

    sbase = s * ROWS_T
    pltpu.sync_copy(zacc_h, acc_sh.at[pl.ds(sbase, ROWS_T)])
    pltpu.sync_copy(zdeg_h, deg_sh.at[pl.ds(sbase, ROWS_T)])
    for i in range(CH // 16):
        ones_v[pl.ds(i * 16, 16)] = jnp.full((16,), 1.0, dtype=_F32)

    nbase = wid * NODES_W

    pltpu.sync_copy(obj_id_h.at[pl.ds(nbase, NODES_W)], nidx)
    pltpu.sync_copy(obj_hash_id_h.at[pl.ds(nbase, NODES_W)], nhidx)
    for k in range(8):
        pltpu.sync_copy(attr_idT_h.at[pl.ds(k * NP + nbase, NODES_W)],
                        aidx.at[pl.ds(k * NODES_W, NODES_W)])
        pltpu.sync_copy(attr_hash_idT_h.at[pl.ds(k * NP + nbase, NODES_W)],
                        ahidx.at[pl.ds(k * NODES_W, NODES_W)])

    def stage_pass(table_h, idxbuf, idx_off, stage, out_h, out_off, wsem,
                   wprev):
        hs = [pltpu.async_copy(
                  table_h.at[idxbuf.at[pl.ds(idx_off + b * CH, CH)]],
                  stage.at[pl.ds(b * CH, CH)], nsem)
              for b in range(NSTAGE // CH)]
        if wprev is not None:
            wprev.wait()
        for h in hs:
            h.wait()
        return pltpu.async_copy(stage, out_h.at[pl.ds(out_off, NSTAGE)], wsem)

    wA = wB = wC = None
    for hh in range(2):
        wA = stage_pass(obj_emb_h, nidx, hh * NSTAGE, rA,
                        o_rows_h, nbase + hh * NSTAGE, wsemA, wA)
        wC = stage_pass(obj_hash_emb_h, nhidx, hh * NSTAGE, rC,
                        oh_rows_h, nbase + hh * NSTAGE, wsemC, wC)
    for k in range(8):
        for hh in range(2):
            wB = stage_pass(attr_emb_h, aidx, k * NODES_W + hh * NSTAGE, rB,
                            av_rows_h, k * NP + nbase + hh * NSTAGE, wsemB, wB)
            wC = stage_pass(attr_hash_emb_h, ahidx,
                            k * NODES_W + hh * NSTAGE, rC,
                            ah_rows_h, k * NP + nbase + hh * NSTAGE, wsemC, wC)
    wA.wait()
    wB.wait()
    wC.wait()

    plsc.subcore_barrier()

    ebase = c * E + s * EDGES_T
    dbase = s * EDGES_T
    GCH = RING * CH
    hblk = (hblk0, hblk1)

    pltpu.sync_copy(ehash2_h.at[pl.ds(ebase, GCH)], hblk[0])
    for b in range(RING):
        pltpu.async_copy(gtab_h.at[hblk[0].at[pl.ds(b * CH, CH)]],
                         rows[b], gsem[b])
        pltpu.async_copy(dst_h.at[pl.ds(dbase + b * CH, CH)], dstb[b],
                         dsem[b])
    pltpu.async_copy(ehash2_h.at[pl.ds(ebase + GCH, GCH)], hblk[1], hsem)

    def esuper(sg, carry):
        for gg in range(2):
            g = sg * 2 + gg
            for b in range(RING):
                pltpu.make_async_copy(gtab_h.at[pl.ds(0, CH)], rows[b],
                                      gsem[b]).wait()
                pltpu.make_async_copy(dst_h.at[pl.ds(0, CH)], dstb[b],
                                      dsem[b]).wait()
                pltpu.async_copy(rows[b], acc_sh.at[dstb[b]], ssem[b],
                                 add=True)

                @pl.when(c == gg)
                def _():
                    pltpu.sync_copy(ones_v, deg_sh.at[dstb[b]], add=True)

            @pl.when(g < EGROUPS - 1)
            def _():
                nxt = hblk[1 - gg]
                pltpu.make_async_copy(ehash2_h.at[pl.ds(0, GCH)], nxt,
                                      hsem).wait()
                for b in range(RING):
                    off2 = (g + 1) * GCH + b * CH
                    pltpu.make_async_copy(rows[b], acc_sh.at[pl.ds(0, CH)],
                                          ssem[b]).wait()
                    pltpu.async_copy(gtab_h.at[nxt.at[pl.ds(b * CH, CH)]],
                                     rows[b], gsem[b])
                    pltpu.async_copy(dst_h.at[pl.ds(dbase + off2, CH)],
                                     dstb[b], dsem[b])

                @pl.when(g < EGROUPS - 2)
                def _():
                    pltpu.async_copy(
                        ehash2_h.at[pl.ds(ebase + (g + 2) * GCH, GCH)],
                        hblk[gg], hsem)
        return carry

    lax.fori_loop(0, EGROUPS // 2, esuper, 0)

    for b in range(RING):
        pltpu.make_async_copy(rows[b], acc_sh.at[pl.ds(0, CH)],
                              ssem[b]).wait()

    plsc.subcore_barrier()

    pltpu.sync_copy(acc_sh.at[pl.ds(sbase, ROWS_T)],
                    acc_out_h.at[pl.ds(c * NP + sbase, ROWS_T)])
    pltpu.sync_copy(deg_sh.at[pl.ds(sbase, ROWS_T)],
                    deg_out_h.at[pl.ds(c * NP + sbase, ROWS_T)])


@functools.lru_cache(maxsize=1)
def _sc_gather_scatter():
    return functools.partial(
        pl.kernel,
        mesh=plsc.VectorSubcoreMesh(core_axis_name="c", subcore_axis_name="s"),
        compiler_params=pltpu.CompilerParams(use_tc_tiling_on_sc=False),
        out_type=(
            jax.ShapeDtypeStruct((NP, OBJ_D), _F32),
            jax.ShapeDtypeStruct((8 * NP, ATTR_DP), _F32),
            jax.ShapeDtypeStruct((NP, HASH_D), _F32),
            jax.ShapeDtypeStruct((8 * NP, HASH_D), _F32),
            jax.ShapeDtypeStruct((NC * NP, HALF_D), _F32),
            jax.ShapeDtypeStruct((NC * NP,), _F32),
        ),
        scratch_types=[
            pltpu.VMEM((NODES_W,), jnp.int32),
            pltpu.VMEM((NODES_W,), jnp.int32),
            pltpu.VMEM((8 * NODES_W,), jnp.int32),
            pltpu.VMEM((8 * NODES_W,), jnp.int32),
        ] + [pltpu.VMEM((CH,), jnp.int32)] * RING
          + [pltpu.VMEM((CH,), jnp.int32)] * RING
          + [pltpu.VMEM((CH, HALF_D), _F32)] * RING
          + [
            pltpu.VMEM((NSTAGE, OBJ_D), _F32),
            pltpu.VMEM((NSTAGE, ATTR_DP), _F32),
            pltpu.VMEM((NSTAGE, HASH_D), _F32),
            pltpu.VMEM((CH,), _F32),
            pltpu.VMEM((RING * CH,), jnp.int32),
            pltpu.VMEM((RING * CH,), jnp.int32),
            pltpu.VMEM_SHARED((NP, HALF_D), _F32),
            pltpu.VMEM_SHARED((NP,), _F32),
            pltpu.SemaphoreType.DMA,
            pltpu.SemaphoreType.DMA,
            pltpu.SemaphoreType.DMA,
            pltpu.SemaphoreType.DMA,
            pltpu.SemaphoreType.DMA,
        ] + [pltpu.SemaphoreType.DMA] * RING
          + [pltpu.SemaphoreType.DMA] * RING
          + [pltpu.SemaphoreType.DMA] * RING,
    )(_sc_body)


def _gtab_body(x_ref, w_ref, b_ref, o_ref):
    o_ref[...] = _gelu(
        jnp.dot(x_ref[...], w_ref[...], preferred_element_type=_F32)
        + b_ref[0])


def _node_body(aidT_ref, bbox_ref, o_ref, av_ref, oh_ref, ah_ref,
               acc_ref, deg_ref,
               w_o_ref, w_a_ref, w_oh_ref, w_ah_ref, w_bx_ref, b1_ref,
               w2_ref, b2_ref, bw1_ref, bb1_ref, bw2_ref, bb2_ref,
               rw2a_ref, rw2b_ref, rb2_ref, g_ref, b_ref, out_ref):
    aid = aidT_ref[...]
    denom = jnp.zeros((aid.shape[1], 1), _F32)
    av = jnp.zeros((aid.shape[1], ATTR_DP), _F32)
    ah = jnp.zeros((aid.shape[1], HASH_D), _F32)
    for k in range(8):
        m = (aid[k] != 0).astype(_F32)[:, None]
        denom = denom + m
        av = av + av_ref[k] * m
        ah = ah + ah_ref[k] * m
    denom = jnp.maximum(denom, 1.0)
    av = av / denom
    ah = ah / denom

    bbox = bbox_ref[...]
    x = bbox[:, 0:1]
    y = bbox[:, 1:2]
    w = bbox[:, 2:3]
    h = bbox[:, 3:4]
    cx = x + 0.5 * w
    cy = y + 0.5 * h
    area = jnp.maximum(w * h, 0.0)
    aspect = w / jnp.maximum(h, 1e-06)
    feats = (x, y, w, h, cx, cy, area, aspect)
    pre = jnp.broadcast_to(bb1_ref[...], (bbox.shape[0], bw1_ref.shape[1]))
    for j, f in enumerate(feats):
        pre = pre + f * bw1_ref[j:j + 1, :]
    z = _gelu(pre)
    z = _gelu(jnp.dot(z, bw2_ref[...], preferred_element_type=_F32)
              + bb2_ref[...])

    base1 = (jnp.dot(o_ref[...], w_o_ref[...], preferred_element_type=_F32)
             + jnp.dot(av, w_a_ref[...], preferred_element_type=_F32)
             + jnp.dot(oh_ref[...], w_oh_ref[...], preferred_element_type=_F32)
             + jnp.dot(ah, w_ah_ref[...], preferred_element_type=_F32)
             + jnp.dot(z, w_bx_ref[...], preferred_element_type=_F32)
             + b1_ref[...])
    base = (jnp.dot(_gelu(base1), w2_ref[...], preferred_element_type=_F32)
            + b2_ref[...])

    degs = (deg_ref[0] + deg_ref[1])[:, None]
    rel_pre = (jnp.dot(acc_ref[0], rw2a_ref[...], preferred_element_type=_F32)
               + jnp.dot(acc_ref[1], rw2b_ref[...],
                         preferred_element_type=_F32)
               + degs * rb2_ref[...])
    rel = rel_pre / jnp.maximum(degs, 1.0)

    hh = base + rel
    mu = jnp.mean(hh, axis=-1, keepdims=True)
    var = jnp.mean((hh - mu) ** 2, axis=-1, keepdims=True)
    out_ref[...] = (hh - mu) * lax.rsqrt(var + 1e-05) * g_ref[...] + b_ref[...]


def kernel(obj_id, attr_id, bbox, obj_hash_id, attr_hash_id, edge_index,
           edge_pred_id, edge_pred_hash_id, obj_emb, attr_emb, obj_hash_emb,
           attr_hash_emb, rel_hash_emb, bbox_w1, bbox_b1, bbox_w2, bbox_b2,
           proj_w1, proj_b1, proj_w2, proj_b2, rel_w1, rel_b1, rel_w2,
           rel_b2, ln_g, ln_b):
    i32 = jnp.int32
    pad_n = NP - N

    obj_id_p = jnp.pad(obj_id.astype(i32), (0, pad_n))
    obj_hash_id_p = jnp.pad(obj_hash_id.astype(i32), (0, pad_n))
    attr_idT = jnp.pad(attr_id.astype(i32).T, ((0, 0), (0, pad_n)))
    attr_hash_idT = jnp.pad(attr_hash_id.astype(i32).T, ((0, 0), (0, pad_n)))
    dst = edge_index[1].astype(i32)
    ehash = edge_pred_hash_id.astype(i32)
    ehash2 = jnp.concatenate([ehash, ehash + HB])
    attr_emb_p = jnp.pad(attr_emb, ((0, 0), (0, ATTR_DP - attr_emb.shape[1])))
    bbox_p = jnp.pad(bbox, ((0, pad_n), (0, 0)))

    rw1s = jnp.concatenate([rel_w1[:, :HALF_D], rel_w1[:, HALF_D:]], axis=0)
    rb1s = jnp.stack([rel_b1[:HALF_D], rel_b1[HALF_D:]])[:, None, :]
    gtab = pl.pallas_call(
        _gtab_body,
        grid=(2, 16),
        in_specs=[
            pl.BlockSpec((HB // 16, HASH_D), lambda h, i: (i, 0)),
            pl.BlockSpec((HASH_D, HALF_D), lambda h, i: (h, 0)),
            pl.BlockSpec((1, 1, HALF_D), lambda h, i: (h, 0, 0)),
        ],
        out_specs=pl.BlockSpec((HB // 16, HALF_D),
                               lambda h, i: (h * 16 + i, 0)),
        out_shape=jax.ShapeDtypeStruct((2 * HB, HALF_D), _F32),
    )(rel_hash_emb, rw1s, rb1s)

    zacc = jnp.zeros((ROWS_T, HALF_D), _F32)
    zdeg = jnp.zeros((ROWS_T,), _F32)

    o_rows, av_rows, oh_rows, ah_rows, acc, deg = _sc_gather_scatter()(
        obj_id_p, obj_hash_id_p, attr_idT.reshape(-1),
        attr_hash_idT.reshape(-1), dst, ehash2,
        obj_emb, attr_emb_p, obj_hash_emb, attr_hash_emb, gtab, zacc, zdeg)
    av_rows = av_rows.reshape(8, NP, ATTR_DP)
    ah_rows = ah_rows.reshape(8, NP, HASH_D)
    acc = acc.reshape(NC, NP, HALF_D)

    w_o = proj_w1[0:80]
    w_a = jnp.pad(proj_w1[80:136], ((0, ATTR_DP - 56), (0, 0)))
    w_oh = proj_w1[136:168]
    w_ah = proj_w1[168:200]
    w_bx = proj_w1[200:248]

    B = 1024
    deg3 = deg.reshape(NC, NP)
    grid = (NP // B,)
    out = pl.pallas_call(
        _node_body,
        grid=grid,
        in_specs=[
            pl.BlockSpec((8, B), lambda i: (0, i)),
            pl.BlockSpec((B, 4), lambda i: (i, 0)),
            pl.BlockSpec((B, OBJ_D), lambda i: (i, 0)),
            pl.BlockSpec((8, B, ATTR_DP), lambda i: (0, i, 0)),
            pl.BlockSpec((B, HASH_D), lambda i: (i, 0)),
            pl.BlockSpec((8, B, HASH_D), lambda i: (0, i, 0)),
            pl.BlockSpec((NC, B, HALF_D), lambda i: (0, i, 0)),
            pl.BlockSpec((NC, B), lambda i: (0, i)),
            pl.BlockSpec((OBJ_D, OUT_D), lambda i: (0, 0)),
            pl.BlockSpec((ATTR_DP, OUT_D), lambda i: (0, 0)),
            pl.BlockSpec((HASH_D, OUT_D), lambda i: (0, 0)),
            pl.BlockSpec((HASH_D, OUT_D), lambda i: (0, 0)),
            pl.BlockSpec((48, OUT_D), lambda i: (0, 0)),
            pl.BlockSpec((1, OUT_D), lambda i: (0, 0)),
            pl.BlockSpec((OUT_D, OUT_D), lambda i: (0, 0)),
            pl.BlockSpec((1, OUT_D), lambda i: (0, 0)),
            pl.BlockSpec((8, 48), lambda i: (0, 0)),
            pl.BlockSpec((1, 48), lambda i: (0, 0)),
            pl.BlockSpec((48, 48), lambda i: (0, 0)),
            pl.BlockSpec((1, 48), lambda i: (0, 0)),
            pl.BlockSpec((HALF_D, OUT_D), lambda i: (0, 0)),
            pl.BlockSpec((HALF_D, OUT_D), lambda i: (0, 0)),
            pl.BlockSpec((1, OUT_D), lambda i: (0, 0)),
            pl.BlockSpec((1, OUT_D), lambda i: (0, 0)),
            pl.BlockSpec((1, OUT_D), lambda i: (0, 0)),
        ],
        out_specs=pl.BlockSpec((B, OUT_D), lambda i: (i, 0)),
        out_shape=jax.ShapeDtypeStruct((NP, OUT_D), _F32),
    )(attr_idT, bbox_p, o_rows, av_rows, oh_rows, ah_rows, acc, deg3,
      w_o, w_a, w_oh, w_ah, w_bx, proj_b1.reshape(1, OUT_D),
      proj_w2, proj_b2.reshape(1, OUT_D),
      bbox_w1, bbox_b1.reshape(1, 48), bbox_w2, bbox_b2.reshape(1, 48),
      rel_w2[:HALF_D], rel_w2[HALF_D:], rel_b2.reshape(1, OUT_D),
      ln_g.reshape(1, OUT_D), ln_b.reshape(1, OUT_D))

    return out[:N]


# speedup vs baseline: 1.0054x; 1.0054x over previous
import functools

import jax
import jax.numpy as jnp
from jax import lax
from jax.experimental import pallas as pl
from jax.experimental.pallas import tpu as pltpu
from jax.experimental.pallas import tpu_sc as plsc

N = 10000
NP = 10240
E = 320000
HB = 65536
OBJ_D = 80
ATTR_DP = 64
HASH_D = 32
OUT_D = 128
HALF_D = 64

NC = 2
NS = 16
NW = NC * NS
NODES_W = NP // NW
EDGES_T = E // NS
CH = 80
ROWS_T = NP // NS
RING = 5
EGROUPS = EDGES_T // (CH * RING)
NSTAGE = 160

_F32 = jnp.float32


def _gelu(x):
    return x * 0.5 * (1.0 + lax.erf(x * 0.7071067811865476))


def _sc_body(obj_id_h, obj_hash_id_h, attr_idT_h, attr_hash_idT_h,
             dst_h, ehash2_h,
             obj_emb_h, attr_emb_h, obj_hash_emb_h, attr_hash_emb_h, gtab_h,
             zacc_h, zdeg_h,
             o_rows_h, av_rows_h, oh_rows_h, ah_rows_h, acc_out_h, deg_out_h,
             nidx, nhidx, aidx, ahidx,
             hidx0, hidx1, hidx2, hidx3, hidx4,
             dstb0, dstb1, dstb2, dstb3, dstb4,
             rows0, rows1, rows2, rows3, rows4,
             rA, rB, rC, ones_v, hblk0, hblk1, acc_sh, deg_sh,
             nsem, wsemA, wsemB, wsemC, hsem,
             gsem0, gsem1, gsem2, gsem3, gsem4,
             dsem0, dsem1, dsem2, dsem3, dsem4,
             ssem0, ssem1, ssem2, ssem3, ssem4,
             qsem0, qsem1, qsem2, qsem3, qsem4):
    c = lax.axis_index("c")
    s = lax.axis_index("s")
    wid = c * NS + s
    hidx = (hidx0, hidx1, hidx2, hidx3, hidx4)
    dstb = (dstb0, dstb1, dstb2, dstb3, dstb4)
    rows = (rows0, rows1, rows2, rows3, rows4)
    gsem = (gsem0, gsem1, gsem2, gsem3, gsem4)
    dsem = (dsem0, dsem1, dsem2, dsem3, dsem4)
    ssem = (ssem0, ssem1, ssem2, ssem3, ssem4)
    qsem = (qsem0, qsem1, qsem2, qsem3, qsem4)

    sbase = s * ROWS_T
    pltpu.sync_copy(zacc_h, acc_sh.at[pl.ds(sbase, ROWS_T)])
    pltpu.sync_copy(zdeg_h, deg_sh.at[pl.ds(sbase, ROWS_T)])
    for i in range(CH // 16):
        ones_v[pl.ds(i * 16, 16)] = jnp.full((16,), 1.0, dtype=_F32)

    nbase = wid * NODES_W

    pltpu.sync_copy(obj_id_h.at[pl.ds(nbase, NODES_W)], nidx)
    pltpu.sync_copy(obj_hash_id_h.at[pl.ds(nbase, NODES_W)], nhidx)
    for k in range(8):
        pltpu.sync_copy(attr_idT_h.at[pl.ds(k * NP + nbase, NODES_W)],
                        aidx.at[pl.ds(k * NODES_W, NODES_W)])
        pltpu.sync_copy(attr_hash_idT_h.at[pl.ds(k * NP + nbase, NODES_W)],
                        ahidx.at[pl.ds(k * NODES_W, NODES_W)])

    def stage_pass(table_h, idxbuf, idx_off, stage, out_h, out_off, wsem,
                   wprev):
        hs = [pltpu.async_copy(
                  table_h.at[idxbuf.at[pl.ds(idx_off + b * CH, CH)]],
                  stage.at[pl.ds(b * CH, CH)], nsem)
              for b in range(NSTAGE // CH)]
        if wprev is not None:
            wprev.wait()
        for h in hs:
            h.wait()
        return pltpu.async_copy(stage, out_h.at[pl.ds(out_off, NSTAGE)], wsem)

    wA = wB = wC = None
    for hh in range(2):
        wA = stage_pass(obj_emb_h, nidx, hh * NSTAGE, rA,
                        o_rows_h, nbase + hh * NSTAGE, wsemA, wA)
        wC = stage_pass(obj_hash_emb_h, nhidx, hh * NSTAGE, rC,
                        oh_rows_h, nbase + hh * NSTAGE, wsemC, wC)
    for k in range(8):
        for hh in range(2):
            wB = stage_pass(attr_emb_h, aidx, k * NODES_W + hh * NSTAGE, rB,
                            av_rows_h, k * NP + nbase + hh * NSTAGE, wsemB, wB)
            wC = stage_pass(attr_hash_emb_h, ahidx,
                            k * NODES_W + hh * NSTAGE, rC,
                            ah_rows_h, k * NP + nbase + hh * NSTAGE, wsemC, wC)
    wA.wait()
    wB.wait()
    wC.wait()

    plsc.subcore_barrier()

    ebase = c * E + s * EDGES_T
    dbase = s * EDGES_T
    GCH = RING * CH
    hblk = (hblk0, hblk1)

    pltpu.sync_copy(ehash2_h.at[pl.ds(ebase, GCH)], hblk[0])
    for b in range(RING):
        pltpu.async_copy(gtab_h.at[hblk[0].at[pl.ds(b * CH, CH)]],
                         rows[b], gsem[b])
        pltpu.async_copy(dst_h.at[pl.ds(dbase + b * CH, CH)], dstb[b],
                         dsem[b])
    pltpu.async_copy(ehash2_h.at[pl.ds(ebase + GCH, GCH)], hblk[1], hsem)

    def esuper(sg, carry):
        for gg in range(2):
            g = sg * 2 + gg
            for b in range(RING):
                pltpu.make_async_copy(gtab_h.at[pl.ds(0, CH)], rows[b],
                                      gsem[b]).wait()
                pltpu.make_async_copy(dst_h.at[pl.ds(0, CH)], dstb[b],
                                      dsem[b]).wait()
                pltpu.async_copy(rows[b], acc_sh.at[dstb[b]], ssem[b],
                                 add=True)

                @pl.when(c == gg)
                def _():
                    @pl.when(g >= 2)
                    def _():
                        pltpu.make_async_copy(
                            ones_v, deg_sh.at[pl.ds(0, CH)], qsem[b]).wait()
                    pltpu.async_copy(ones_v, deg_sh.at[dstb[b]], qsem[b],
                                     add=True)

            @pl.when(g < EGROUPS - 1)
            def _():
                nxt = hblk[1 - gg]
                pltpu.make_async_copy(ehash2_h.at[pl.ds(0, GCH)], nxt,
                                      hsem).wait()
                for b in range(RING):
                    off2 = (g + 1) * GCH + b * CH
                    pltpu.make_async_copy(rows[b], acc_sh.at[pl.ds(0, CH)],
                                          ssem[b]).wait()
                    pltpu.async_copy(gtab_h.at[nxt.at[pl.ds(b * CH, CH)]],
                                     rows[b], gsem[b])
                    pltpu.async_copy(dst_h.at[pl.ds(dbase + off2, CH)],
                                     dstb[b], dsem[b])

                @pl.when(g < EGROUPS - 2)
                def _():
                    pltpu.async_copy(
                        ehash2_h.at[pl.ds(ebase + (g + 2) * GCH, GCH)],
                        hblk[gg], hsem)
        return carry

    lax.fori_loop(0, EGROUPS // 2, esuper, 0)

    for b in range(RING):
        pltpu.make_async_copy(rows[b], acc_sh.at[pl.ds(0, CH)],
                              ssem[b]).wait()
        pltpu.make_async_copy(ones_v, deg_sh.at[pl.ds(0, CH)],
                              qsem[b]).wait()

    plsc.subcore_barrier()

    pltpu.sync_copy(acc_sh.at[pl.ds(sbase, ROWS_T)],
                    acc_out_h.at[pl.ds(c * NP + sbase, ROWS_T)])
    pltpu.sync_copy(deg_sh.at[pl.ds(sbase, ROWS_T)],
                    deg_out_h.at[pl.ds(c * NP + sbase, ROWS_T)])


@functools.lru_cache(maxsize=1)
def _sc_gather_scatter():
    return functools.partial(
        pl.kernel,
        mesh=plsc.VectorSubcoreMesh(core_axis_name="c", subcore_axis_name="s"),
        compiler_params=pltpu.CompilerParams(use_tc_tiling_on_sc=False),
        out_type=(
            jax.ShapeDtypeStruct((NP, OBJ_D), _F32),
            jax.ShapeDtypeStruct((8 * NP, ATTR_DP), _F32),
            jax.ShapeDtypeStruct((NP, HASH_D), _F32),
            jax.ShapeDtypeStruct((8 * NP, HASH_D), _F32),
            jax.ShapeDtypeStruct((NC * NP, HALF_D), _F32),
            jax.ShapeDtypeStruct((NC * NP,), _F32),
        ),
        scratch_types=[
            pltpu.VMEM((NODES_W,), jnp.int32),
            pltpu.VMEM((NODES_W,), jnp.int32),
            pltpu.VMEM((8 * NODES_W,), jnp.int32),
            pltpu.VMEM((8 * NODES_W,), jnp.int32),
        ] + [pltpu.VMEM((CH,), jnp.int32)] * RING
          + [pltpu.VMEM((CH,), jnp.int32)] * RING
          + [pltpu.VMEM((CH, HALF_D), _F32)] * RING
          + [
            pltpu.VMEM((NSTAGE, OBJ_D), _F32),
            pltpu.VMEM((NSTAGE, ATTR_DP), _F32),
            pltpu.VMEM((NSTAGE, HASH_D), _F32),
            pltpu.VMEM((CH,), _F32),
            pltpu.VMEM((RING * CH,), jnp.int32),
            pltpu.VMEM((RING * CH,), jnp.int32),
            pltpu.VMEM_SHARED((NP, HALF_D), _F32),
            pltpu.VMEM_SHARED((NP,), _F32),
            pltpu.SemaphoreType.DMA,
            pltpu.SemaphoreType.DMA,
            pltpu.SemaphoreType.DMA,
            pltpu.SemaphoreType.DMA,
            pltpu.SemaphoreType.DMA,
        ] + [pltpu.SemaphoreType.DMA] * RING
          + [pltpu.SemaphoreType.DMA] * RING
          + [pltpu.SemaphoreType.DMA] * RING
          + [pltpu.SemaphoreType.DMA] * RING,
    )(_sc_body)


def _gtab_body(x_ref, w_ref, b_ref, o_ref):
    o_ref[...] = _gelu(
        jnp.dot(x_ref[...], w_ref[...], preferred_element_type=_F32)
        + b_ref[0])


def _node_body(aidT_ref, bbox_ref, o_ref, av_ref, oh_ref, ah_ref,
               acc_ref, deg_ref,
               w_o_ref, w_a_ref, w_oh_ref, w_ah_ref, w_bx_ref, b1_ref,
               w2_ref, b2_ref, bw1_ref, bb1_ref, bw2_ref, bb2_ref,
               rw2a_ref, rw2b_ref, rb2_ref, g_ref, b_ref, out_ref):
    aid = aidT_ref[...]
    denom = jnp.zeros((aid.shape[1], 1), _F32)
    av = jnp.zeros((aid.shape[1], ATTR_DP), _F32)
    ah = jnp.zeros((aid.shape[1], HASH_D), _F32)
    for k in range(8):
        m = (aid[k] != 0).astype(_F32)[:, None]
        denom = denom + m
        av = av + av_ref[k] * m
        ah = ah + ah_ref[k] * m
    denom = jnp.maximum(denom, 1.0)
    av = av / denom
    ah = ah / denom

    bbox = bbox_ref[...]
    x = bbox[:, 0:1]
    y = bbox[:, 1:2]
    w = bbox[:, 2:3]
    h = bbox[:, 3:4]
    cx = x + 0.5 * w
    cy = y + 0.5 * h
    area = jnp.maximum(w * h, 0.0)
    aspect = w / jnp.maximum(h, 1e-06)
    feats = (x, y, w, h, cx, cy, area, aspect)
    pre = jnp.broadcast_to(bb1_ref[...], (bbox.shape[0], bw1_ref.shape[1]))
    for j, f in enumerate(feats):
        pre = pre + f * bw1_ref[j:j + 1, :]
    z = _gelu(pre)
    z = _gelu(jnp.dot(z, bw2_ref[...], preferred_element_type=_F32)
              + bb2_ref[...])

    base1 = (jnp.dot(o_ref[...], w_o_ref[...], preferred_element_type=_F32)
             + jnp.dot(av, w_a_ref[...], preferred_element_type=_F32)
             + jnp.dot(oh_ref[...], w_oh_ref[...], preferred_element_type=_F32)
             + jnp.dot(ah, w_ah_ref[...], preferred_element_type=_F32)
             + jnp.dot(z, w_bx_ref[...], preferred_element_type=_F32)
             + b1_ref[...])
    base = (jnp.dot(_gelu(base1), w2_ref[...], preferred_element_type=_F32)
            + b2_ref[...])

    degs = (deg_ref[0] + deg_ref[1])[:, None]
    rel_pre = (jnp.dot(acc_ref[0], rw2a_ref[...], preferred_element_type=_F32)
               + jnp.dot(acc_ref[1], rw2b_ref[...],
                         preferred_element_type=_F32)
               + degs * rb2_ref[...])
    rel = rel_pre / jnp.maximum(degs, 1.0)

    hh = base + rel
    mu = jnp.mean(hh, axis=-1, keepdims=True)
    var = jnp.mean((hh - mu) ** 2, axis=-1, keepdims=True)
    out_ref[...] = (hh - mu) * lax.rsqrt(var + 1e-05) * g_ref[...] + b_ref[...]


def kernel(obj_id, attr_id, bbox, obj_hash_id, attr_hash_id, edge_index,
           edge_pred_id, edge_pred_hash_id, obj_emb, attr_emb, obj_hash_emb,
           attr_hash_emb, rel_hash_emb, bbox_w1, bbox_b1, bbox_w2, bbox_b2,
           proj_w1, proj_b1, proj_w2, proj_b2, rel_w1, rel_b1, rel_w2,
           rel_b2, ln_g, ln_b):
    i32 = jnp.int32
    pad_n = NP - N

    obj_id_p = jnp.pad(obj_id.astype(i32), (0, pad_n))
    obj_hash_id_p = jnp.pad(obj_hash_id.astype(i32), (0, pad_n))
    attr_idT = jnp.pad(attr_id.astype(i32).T, ((0, 0), (0, pad_n)))
    attr_hash_idT = jnp.pad(attr_hash_id.astype(i32).T, ((0, 0), (0, pad_n)))
    dst = edge_index[1].astype(i32)
    ehash = edge_pred_hash_id.astype(i32)
    ehash2 = jnp.concatenate([ehash, ehash + HB])
    attr_emb_p = jnp.pad(attr_emb, ((0, 0), (0, ATTR_DP - attr_emb.shape[1])))
    bbox_p = jnp.pad(bbox, ((0, pad_n), (0, 0)))

    rw1s = jnp.concatenate([rel_w1[:, :HALF_D], rel_w1[:, HALF_D:]], axis=0)
    rb1s = jnp.stack([rel_b1[:HALF_D], rel_b1[HALF_D:]])[:, None, :]
    gtab = pl.pallas_call(
        _gtab_body,
        grid=(2, 16),
        in_specs=[
            pl.BlockSpec((HB // 16, HASH_D), lambda h, i: (i, 0)),
            pl.BlockSpec((HASH_D, HALF_D), lambda h, i: (h, 0)),
            pl.BlockSpec((1, 1, HALF_D), lambda h, i: (h, 0, 0)),
        ],
        out_specs=pl.BlockSpec((HB // 16, HALF_D),
                               lambda h, i: (h * 16 + i, 0)),
        out_shape=jax.ShapeDtypeStruct((2 * HB, HALF_D), _F32),
    )(rel_hash_emb, rw1s, rb1s)

    zacc = jnp.zeros((ROWS_T, HALF_D), _F32)
    zdeg = jnp.zeros((ROWS_T,), _F32)

    o_rows, av_rows, oh_rows, ah_rows, acc, deg = _sc_gather_scatter()(
        obj_id_p, obj_hash_id_p, attr_idT.reshape(-1),
        attr_hash_idT.reshape(-1), dst, ehash2,
        obj_emb, attr_emb_p, obj_hash_emb, attr_hash_emb, gtab, zacc, zdeg)
    av_rows = av_rows.reshape(8, NP, ATTR_DP)
    ah_rows = ah_rows.reshape(8, NP, HASH_D)
    acc = acc.reshape(NC, NP, HALF_D)

    w_o = proj_w1[0:80]
    w_a = jnp.pad(proj_w1[80:136], ((0, ATTR_DP - 56), (0, 0)))
    w_oh = proj_w1[136:168]
    w_ah = proj_w1[168:200]
    w_bx = proj_w1[200:248]

    B = 1024
    deg3 = deg.reshape(NC, NP)
    grid = (NP // B,)
    out = pl.pallas_call(
        _node_body,
        grid=grid,
        in_specs=[
            pl.BlockSpec((8, B), lambda i: (0, i)),
            pl.BlockSpec((B, 4), lambda i: (i, 0)),
            pl.BlockSpec((B, OBJ_D), lambda i: (i, 0)),
            pl.BlockSpec((8, B, ATTR_DP), lambda i: (0, i, 0)),
            pl.BlockSpec((B, HASH_D), lambda i: (i, 0)),
            pl.BlockSpec((8, B, HASH_D), lambda i: (0, i, 0)),
            pl.BlockSpec((NC, B, HALF_D), lambda i: (0, i, 0)),
            pl.BlockSpec((NC, B), lambda i: (0, i)),
            pl.BlockSpec((OBJ_D, OUT_D), lambda i: (0, 0)),
            pl.BlockSpec((ATTR_DP, OUT_D), lambda i: (0, 0)),
            pl.BlockSpec((HASH_D, OUT_D), lambda i: (0, 0)),
            pl.BlockSpec((HASH_D, OUT_D), lambda i: (0, 0)),
            pl.BlockSpec((48, OUT_D), lambda i: (0, 0)),
            pl.BlockSpec((1, OUT_D), lambda i: (0, 0)),
            pl.BlockSpec((OUT_D, OUT_D), lambda i: (0, 0)),
            pl.BlockSpec((1, OUT_D), lambda i: (0, 0)),
            pl.BlockSpec((8, 48), lambda i: (0, 0)),
            pl.BlockSpec((1, 48), lambda i: (0, 0)),
            pl.BlockSpec((48, 48), lambda i: (0, 0)),
            pl.BlockSpec((1, 48), lambda i: (0, 0)),
            pl.BlockSpec((HALF_D, OUT_D), lambda i: (0, 0)),
            pl.BlockSpec((HALF_D, OUT_D), lambda i: (0, 0)),
            pl.BlockSpec((1, OUT_D), lambda i: (0, 0)),
            pl.BlockSpec((1, OUT_D), lambda i: (0, 0)),
            pl.BlockSpec((1, OUT_D), lambda i: (0, 0)),
        ],
        out_specs=pl.BlockSpec((B, OUT_D), lambda i: (i, 0)),
        out_shape=jax.ShapeDtypeStruct((NP, OUT_D), _F32),
    )(attr_idT, bbox_p, o_rows, av_rows, oh_rows, ah_rows, acc, deg3,
      w_o, w_a, w_oh, w_ah, w_bx, proj_b1.reshape(1, OUT_D),
      proj_w2, proj_b2.reshape(1, OUT_D),
      bbox_w1, bbox_b1.reshape(1, 48), bbox_w2, bbox_b2.reshape(1, 48),
      rel_w2[:HALF_D], rel_w2[HALF_D:], rel_b2.reshape(1, OUT_D),
      ln_g.reshape(1, OUT_D), ln_b.reshape(1, OUT_D))

    return out[:N]
